# Initial kernel scaffold; baseline (speedup 1.0000x reference)
import jax
import jax.numpy as jnp
from jax.experimental import pallas as pl


def kernel(nodes, edges, receivers, senders, node_graph_idx, edge_graph_idx, atom_tables, bond_tables, W_edge, b_edge, W_node, b_node, global_table):
    raise NotImplementedError("write your pallas kernel here")


# speedup vs baseline: 5.4209x; 5.4209x over previous
import functools

import jax
import jax.numpy as jnp
from jax import lax
from jax.experimental import pallas as pl
from jax.experimental.pallas import tpu as pltpu
from jax.experimental.pallas import tpu_sc as plsc

N = 10000
E = 320000
D = 128
B = 256
AV = 128
BV = 16
NA = 9
NB = 3

NC = 2
NS = 16
NW = NC * NS

EG = 128
N_EGROUPS = E // EG
N_NGROUPS = -(-N // EG)
NPAD = N_NGROUPS * EG


def _prep_body(at_ref, bt_ref, we_ref, be_ref, wn_ref, bn_ref, gt_ref,
               e0_ref, e1_ref, e2_ref, nt_ref,
               taw_ref, tbc_ref, glat_ref, cidx_ref, nidx_ref):
    wn = wn_ref[...]
    bn = bn_ref[...]
    for f in range(NA):
        r = jnp.dot(at_ref[f], wn, preferred_element_type=jnp.float32)
        if f == 0:
            r = r + bn
        taw_ref[pl.ds(f * AV, AV), :] = r

    we = we_ref[...]
    be = be_ref[...]
    t0 = jnp.dot(bt_ref[0], we, preferred_element_type=jnp.float32) + be
    t1 = jnp.dot(bt_ref[1], we, preferred_element_type=jnp.float32)
    t2 = jnp.dot(bt_ref[2], we, preferred_element_type=jnp.float32)
    for i1 in range(BV):
        t01 = t0 + t1[i1:i1 + 1, :]
        for i2 in range(BV):
            tbc_ref[pl.ds(i2 * 256 + i1 * 16, BV), :] = t01 + t2[i2:i2 + 1, :]

    glat_ref[...] = jnp.broadcast_to(gt_ref[...], (B, D))
    cidx_ref[...] = e0_ref[...] + 16 * e1_ref[...] + 256 * e2_ref[...]
    fld = lax.broadcasted_iota(jnp.int32, (NA * N_NGROUPS, D), 0) // N_NGROUPS
    nidx_ref[...] = nt_ref[...] + AV * fld


def _sc_body(tbc, taw, cidx, nidx, eout, nout,
             eidx_v, erows_v, nidx_v, nacc_v, ngat_v, sem):
    cid = lax.axis_index("c")
    sid = lax.axis_index("s")
    wid = sid * NC + cid

    def edge_group(i, carry):
        base = (wid + i * NW) * EG
        pltpu.sync_copy(cidx.at[pl.ds(base, EG)], eidx_v)
        pltpu.async_copy(tbc.at[eidx_v], erows_v, sem).wait()

        def relu_row(r, c):
            for j in range(D // 16):
                sl = pl.ds(j * 16, 16)
                erows_v[r, sl] = jnp.maximum(erows_v[r, sl], 0.0)
            return c

        lax.fori_loop(0, EG, relu_row, 0)
        pltpu.sync_copy(erows_v, eout.at[pl.ds(base, EG)])
        return carry

    my_eg = (N_EGROUPS - wid + NW - 1) // NW
    lax.fori_loop(0, my_eg, edge_group, 0)

    def node_group(i, carry):
        g = wid + i * NW
        pltpu.sync_copy(nidx.at[pl.ds(g * EG, EG)], nidx_v)
        pltpu.async_copy(taw.at[nidx_v], nacc_v, sem).wait()
        for f in range(1, NA):
            pltpu.sync_copy(nidx.at[pl.ds((f * N_NGROUPS + g) * EG, EG)],
                            nidx_v)
            pltpu.async_copy(taw.at[nidx_v], ngat_v, sem).wait()

            def add_row(r, c):
                for j in range(D // 16):
                    sl = pl.ds(j * 16, 16)
                    nacc_v[r, sl] = nacc_v[r, sl] + ngat_v[r, sl]
                return c

            lax.fori_loop(0, EG, add_row, 0)

        def relu_row(r, c):
            for j in range(D // 16):
                sl = pl.ds(j * 16, 16)
                nacc_v[r, sl] = jnp.maximum(nacc_v[r, sl], 0.0)
            return c

        lax.fori_loop(0, EG, relu_row, 0)
        pltpu.sync_copy(nacc_v, nout.at[pl.ds(g * EG, EG)])
        return carry

    my_ng = (N_NGROUPS - wid + NW - 1) // NW
    lax.fori_loop(0, my_ng, node_group, 0)


def kernel(nodes, edges, receivers, senders, node_graph_idx, edge_graph_idx,
           atom_tables, bond_tables, W_edge, b_edge, W_node, b_node,
           global_table):
    e0 = edges[:, 0].reshape(E // D, D)
    e1 = edges[:, 1].reshape(E // D, D)
    e2 = edges[:, 2].reshape(E // D, D)
    nodes_t = jnp.pad(nodes.T, ((0, 0), (0, NPAD - N))).reshape(
        NA * N_NGROUPS, EG)

    taw, tbc, glat, cidx2d, nidx2d = pl.pallas_call(
        _prep_body,
        out_shape=(
            jax.ShapeDtypeStruct((NA * AV, D), jnp.float32),
            jax.ShapeDtypeStruct((BV * BV * BV, D), jnp.float32),
            jax.ShapeDtypeStruct((B, D), jnp.float32),
            jax.ShapeDtypeStruct((E // D, D), jnp.int32),
            jax.ShapeDtypeStruct((NA * N_NGROUPS, EG), jnp.int32),
        ),
    )(atom_tables, bond_tables, W_edge, b_edge.reshape(1, D),
      W_node, b_node.reshape(1, D), global_table,
      e0, e1, e2, nodes_t)

    cidx = cidx2d.reshape(E)
    nidx = nidx2d.reshape(NA * NPAD)

    mesh = plsc.VectorSubcoreMesh(core_axis_name="c", subcore_axis_name="s",
                                  num_cores=NC, num_subcores=NS)
    sc = functools.partial(
        pl.kernel,
        out_type=(
            jax.ShapeDtypeStruct((E, D), jnp.float32),
            jax.ShapeDtypeStruct((NPAD, D), jnp.float32),
        ),
        mesh=mesh,
        scratch_types=[
            pltpu.VMEM((EG,), jnp.int32),
            pltpu.VMEM((EG, D), jnp.float32),
            pltpu.VMEM((EG,), jnp.int32),
            pltpu.VMEM((EG, D), jnp.float32),
            pltpu.VMEM((EG, D), jnp.float32),
            pltpu.SemaphoreType.DMA,
        ],
    )(_sc_body)

    edges_update, nodes_pad = sc(tbc, taw, cidx, nidx)
    nodes_update = nodes_pad[:N]

    return (nodes_update, edges_update, receivers, senders, glat,
            node_graph_idx, edge_graph_idx)
